# baseline (speedup 1.0000x reference)
import functools

import jax
import jax.numpy as jnp
from jax import lax
from jax.experimental import pallas as pl
from jax.experimental.pallas import tpu as pltpu
from jax.experimental.pallas import tpu_sc as plsc

EMBED = 64
FACTOR = 8.0

NUM_CORES = 2
NUM_SUBCORES = 16
NUM_WORKERS = NUM_CORES * NUM_SUBCORES
LANES = 16
VB = 128

TILED = pltpu.CompilerParams(
    use_tc_tiling_on_sc=True, needs_layout_passes=False
)
LINEAR = pltpu.CompilerParams(
    use_tc_tiling_on_sc=False, needs_layout_passes=False
)

PACK_ROWS = 256


@jax.jit
def _pack_table(table):
    vocab = table.shape[0]
    n_chunks = vocab // PACK_ROWS
    tail = vocab - n_chunks * PACK_ROWS
    chunks = (n_chunks // NUM_WORKERS) & ~1
    n_extra = n_chunks - chunks * NUM_WORKERS
    assert tail % 16 == 0
    mesh = plsc.VectorSubcoreMesh(core_axis_name="c", subcore_axis_name="s")

    @functools.partial(
        pl.kernel,
        out_type=jax.ShapeDtypeStruct((vocab // 2, VB), jnp.float32),
        mesh=mesh,
        scratch_types=[
            pltpu.VMEM((PACK_ROWS, EMBED), jnp.float32),
            pltpu.VMEM((PACK_ROWS, EMBED), jnp.float32),
            pltpu.VMEM((PACK_ROWS // 2, VB), jnp.float32),
            pltpu.VMEM((PACK_ROWS // 2, VB), jnp.float32),
            pltpu.SemaphoreType.DMA,
            pltpu.SemaphoreType.DMA,
            pltpu.SemaphoreType.DMA,
            pltpu.SemaphoreType.DMA,
        ],
        compiler_params=TILED,
    )
    def body(table_hbm, tl_hbm, s0, s1, w0, w1, gs0, gs1, ws0, ws1):
        wid = lax.axis_index("s") * NUM_CORES + lax.axis_index("c")
        row0 = wid * chunks * PACK_ROWS

        def stage(i, sbuf, gsem, nrows=PACK_ROWS):
            pltpu.async_copy(
                table_hbm.at[pl.ds(row0 + i * PACK_ROWS, nrows)],
                sbuf.at[pl.ds(0, nrows)], gsem,
            )

        def drain_stage(sbuf, gsem, nrows=PACK_ROWS):
            pltpu.make_async_copy(
                table_hbm.at[pl.ds(0, nrows)], sbuf.at[pl.ds(0, nrows)], gsem
            ).wait()

        def pack(sbuf, wbuf, nrows=PACK_ROWS):
            def step_m(m, c2):
                for k in range(2 * (EMBED // LANES)):
                    v = sbuf[2 * m + k // 4, pl.ds(16 * (k % 4), 16)]
                    wbuf[m, pl.ds(16 * k, 16)] = v * FACTOR
                return c2

            lax.fori_loop(0, nrows // 2, step_m, 0, unroll=4)

        out_row0 = wid * chunks * (PACK_ROWS // 2)

        def fire_write(i, wbuf, wsem, nrows=PACK_ROWS):
            pltpu.async_copy(
                wbuf.at[pl.ds(0, nrows // 2)],
                tl_hbm.at[pl.ds(out_row0 + i * (PACK_ROWS // 2), nrows // 2)],
                wsem,
            )

        def drain_write(wbuf, wsem, nrows=PACK_ROWS):
            pltpu.make_async_copy(
                wbuf.at[pl.ds(0, nrows // 2)],
                tl_hbm.at[pl.ds(0, nrows // 2)], wsem
            ).wait()

        stage(0, s0, gs0)

        def step(j, carry):
            i0 = 2 * j
            i1 = i0 + 1

            drain_stage(s0, gs0)
            stage(i1, s1, gs1)

            @pl.when(j > 0)
            def _():
                drain_write(w0, ws0)

            pack(s0, w0)
            fire_write(i0, w0, ws0)

            drain_stage(s1, gs1)

            @pl.when(i1 + 1 < chunks)
            def _():
                stage(i1 + 1, s0, gs0)

            @pl.when(j > 0)
            def _():
                drain_write(w1, ws1)

            pack(s1, w1)
            fire_write(i1, w1, ws1)
            return carry

        lax.fori_loop(0, chunks // 2, step, 0)
        drain_write(w0, ws0)
        drain_write(w1, ws1)

        @pl.when(wid < n_extra)
        def _():
            base = chunks * NUM_WORKERS * PACK_ROWS
            i = (base - row0) // PACK_ROWS + wid
            stage(i, s0, gs0)
            drain_stage(s0, gs0)
            pack(s0, w0)
            fire_write(i, w0, ws0)
            drain_write(w0, ws0)

        @pl.when((wid == n_extra) & (tail > 0))
        def _():
            base = n_chunks * PACK_ROWS
            i = (base - row0) // PACK_ROWS
            stage(i, s0, gs0, nrows=tail)
            drain_stage(s0, gs0, nrows=tail)
            pack(s0, w0, nrows=tail)
            fire_write(i, w0, ws0, nrows=tail)
            drain_write(w0, ws0, nrows=tail)

    return body(table)


@jax.jit
def _gather_out(srcT, tableL):
    row_len, n_rows = srcT.shape
    assert n_rows == NUM_WORKERS * VB and row_len % 2 == 0
    mesh = plsc.VectorSubcoreMesh(core_axis_name="c", subcore_axis_name="s")

    @functools.partial(
        pl.kernel,
        out_type=jax.ShapeDtypeStruct(
            (row_len, EMBED // 8, n_rows // VB, 8, VB), jnp.float32),
        mesh=mesh,
        scratch_types=[
            pltpu.VMEM((row_len, VB), jnp.int32),
            pltpu.VMEM((VB, VB), jnp.float32),
            pltpu.VMEM((VB, VB), jnp.float32),
            pltpu.VMEM((EMBED // 8, 8, VB + 1), jnp.float32),
            pltpu.VMEM((EMBED // 8, 8, VB + 1), jnp.float32),
            pltpu.VMEM((VB,), jnp.int32),
            pltpu.VMEM((VB,), jnp.int32),
            pltpu.SemaphoreType.DMA,
            pltpu.SemaphoreType.DMA,
            pltpu.SemaphoreType.DMA,
            pltpu.SemaphoreType.DMA,
            pltpu.SemaphoreType.DMA,
        ],
        compiler_params=LINEAR,
    )
    def body(tableL_hbm, srcT_hbm, out4_hbm, idxT, g0, g1, w0, w1,
             h0, h1, isem, gs0, gs1, ws0, ws1):
        wid = lax.axis_index("s") * NUM_CORES + lax.axis_index("c")
        col0 = wid * VB
        iota = lax.iota(jnp.int32, LANES)

        pltpu.async_copy(srcT_hbm.at[:, pl.ds(col0, VB)], idxT, isem)
        pltpu.make_async_copy(
            srcT_hbm.at[:, pl.ds(0, VB)], idxT, isem
        ).wait()

        r1 = [lax.shift_right_logical(iota + 16 * k, 3) for k in range(4)]
        r2 = [lax.bitwise_and(iota + 16 * k, 7) for k in range(4)]

        def fire_gather(c, hbuf, gbuf, gsem):
            def half(t, c2):
                hbuf[pl.ds(16 * t, 16)] = lax.shift_right_logical(
                    idxT[c, pl.ds(16 * t, 16)], 1
                )
                return c2

            lax.fori_loop(0, 8, half, 0, unroll=8)
            pltpu.async_copy(tableL_hbm.at[hbuf], gbuf, gsem)

        def drain_gather(gbuf, gsem):
            pltpu.make_async_copy(
                tableL_hbm.at[pl.ds(0, VB)], gbuf, gsem
            ).wait()

        def build(c, gbuf, wbuf):
            def step_t(t, c2):
                parv = lax.mul(
                    lax.bitwise_and(idxT[c, pl.ds(16 * t, 16)], 1), EMBED
                )
                for j in range(LANES):
                    b = 16 * t + j
                    off = parv[j]
                    colv = jnp.full((LANES,), b, jnp.int32)
                    for k in range(EMBED // LANES):
                        v = gbuf[b, pl.ds(off + 16 * k, 16)]
                        plsc.store_scatter(wbuf, (r1[k], r2[k], colv), v)
                return c2

            lax.fori_loop(0, 8, step_t, 0)

        def fire_write(c, wbuf, wsem):
            pltpu.async_copy(
                wbuf.at[:, :, pl.ds(0, VB)], out4_hbm.at[c, :, wid], wsem
            )

        def drain_write(wbuf, wsem):
            pltpu.make_async_copy(
                wbuf.at[:, :, pl.ds(0, VB)], out4_hbm.at[0, :, 0], wsem
            ).wait()

        fire_gather(0, h0, g0, gs0)

        def step(j, carry):
            c0 = 2 * j
            c1 = c0 + 1

            drain_gather(g0, gs0)
            fire_gather(c1, h1, g1, gs1)

            @pl.when(j > 0)
            def _():
                drain_write(w0, ws0)

            build(c0, g0, w0)
            fire_write(c0, w0, ws0)

            drain_gather(g1, gs1)

            @pl.when(c1 + 1 < row_len)
            def _():
                fire_gather(c1 + 1, h0, g0, gs0)

            @pl.when(j > 0)
            def _():
                drain_write(w1, ws1)

            build(c1, g1, w1)
            fire_write(c1, w1, ws1)
            return carry

        lax.fori_loop(0, row_len // 2, step, 0)
        drain_write(w0, ws0)
        drain_write(w1, ws1)

    return body(tableL, srcT)


def kernel(src, table):
    n_rows, row_len = src.shape
    vocab, embed = table.shape
    assert embed == EMBED and n_rows == NUM_WORKERS * VB
    tableL = _pack_table(table)
    out4 = _gather_out(src.T, tableL)
    return jnp.reshape(
        jnp.transpose(out4, (2, 4, 0, 1, 3)), (n_rows, row_len, embed)
    )


# speedup vs baseline: 1.1874x; 1.1874x over previous
import functools

import jax
import jax.numpy as jnp
from jax import lax
from jax.experimental import pallas as pl
from jax.experimental.pallas import tpu as pltpu
from jax.experimental.pallas import tpu_sc as plsc

EMBED = 64
FACTOR = 8.0

NUM_CORES = 2
NUM_SUBCORES = 16
NUM_WORKERS = NUM_CORES * NUM_SUBCORES
LANES = 16
VB = 128

TILED = pltpu.CompilerParams(
    use_tc_tiling_on_sc=True, needs_layout_passes=False
)
LINEAR = pltpu.CompilerParams(
    use_tc_tiling_on_sc=False, needs_layout_passes=False
)

@jax.jit
def _gather_out(srcT, tableL):
    row_len, n_rows = srcT.shape
    assert n_rows == NUM_WORKERS * VB and row_len % 2 == 0
    mesh = plsc.VectorSubcoreMesh(core_axis_name="c", subcore_axis_name="s")

    @functools.partial(
        pl.kernel,
        out_type=jax.ShapeDtypeStruct(
            (row_len, EMBED // 8, n_rows // VB, 8, VB), jnp.float32),
        mesh=mesh,
        scratch_types=[
            pltpu.VMEM((row_len, VB), jnp.int32),
            pltpu.VMEM((VB, VB), jnp.float32),
            pltpu.VMEM((VB, VB), jnp.float32),
            pltpu.VMEM((EMBED // 8, 8, VB + 1), jnp.float32),
            pltpu.VMEM((EMBED // 8, 8, VB + 1), jnp.float32),
            pltpu.VMEM((VB,), jnp.int32),
            pltpu.VMEM((VB,), jnp.int32),
            pltpu.SemaphoreType.DMA,
            pltpu.SemaphoreType.DMA,
            pltpu.SemaphoreType.DMA,
            pltpu.SemaphoreType.DMA,
            pltpu.SemaphoreType.DMA,
        ],
        compiler_params=LINEAR,
    )
    def body(tableL_hbm, srcT_hbm, out4_hbm, idxT, g0, g1, w0, w1,
             h0, h1, isem, gs0, gs1, ws0, ws1):
        wid = lax.axis_index("s") * NUM_CORES + lax.axis_index("c")
        col0 = wid * VB
        iota = lax.iota(jnp.int32, LANES)

        pltpu.async_copy(srcT_hbm.at[:, pl.ds(col0, VB)], idxT, isem)
        pltpu.make_async_copy(
            srcT_hbm.at[:, pl.ds(0, VB)], idxT, isem
        ).wait()

        r1 = [lax.shift_right_logical(iota + 16 * k, 3) for k in range(4)]
        r2 = [lax.bitwise_and(iota + 16 * k, 7) for k in range(4)]

        def fire_gather(c, hbuf, gbuf, gsem):
            def half(t, c2):
                hbuf[pl.ds(16 * t, 16)] = lax.shift_right_logical(
                    idxT[c, pl.ds(16 * t, 16)], 1
                )
                return c2

            lax.fori_loop(0, 8, half, 0, unroll=8)
            pltpu.async_copy(tableL_hbm.at[hbuf], gbuf, gsem)

        def drain_gather(gbuf, gsem):
            pltpu.make_async_copy(
                tableL_hbm.at[pl.ds(0, VB)], gbuf, gsem
            ).wait()

        def build(c, gbuf, wbuf):
            def step_t(t, c2):
                parv = lax.mul(
                    lax.bitwise_and(idxT[c, pl.ds(16 * t, 16)], 1), EMBED
                )
                for j in range(LANES):
                    b = 16 * t + j
                    off = parv[j]
                    colv = jnp.full((LANES,), b, jnp.int32)
                    for k in range(EMBED // LANES):
                        v = gbuf[b, pl.ds(off + 16 * k, 16)]
                        plsc.store_scatter(wbuf, (r1[k], r2[k], colv), v * FACTOR)
                return c2

            lax.fori_loop(0, 8, step_t, 0)

        def fire_write(c, wbuf, wsem):
            pltpu.async_copy(
                wbuf.at[:, :, pl.ds(0, VB)], out4_hbm.at[c, :, wid], wsem
            )

        def drain_write(wbuf, wsem):
            pltpu.make_async_copy(
                wbuf.at[:, :, pl.ds(0, VB)], out4_hbm.at[0, :, 0], wsem
            ).wait()

        fire_gather(0, h0, g0, gs0)

        def step(j, carry):
            c0 = 2 * j
            c1 = c0 + 1

            drain_gather(g0, gs0)
            fire_gather(c1, h1, g1, gs1)

            @pl.when(j > 0)
            def _():
                drain_write(w0, ws0)

            build(c0, g0, w0)
            fire_write(c0, w0, ws0)

            drain_gather(g1, gs1)

            @pl.when(c1 + 1 < row_len)
            def _():
                fire_gather(c1 + 1, h0, g0, gs0)

            @pl.when(j > 0)
            def _():
                drain_write(w1, ws1)

            build(c1, g1, w1)
            fire_write(c1, w1, ws1)
            return carry

        lax.fori_loop(0, row_len // 2, step, 0)
        drain_write(w0, ws0)
        drain_write(w1, ws1)

    return body(tableL, srcT)


def kernel(src, table):
    n_rows, row_len = src.shape
    vocab, embed = table.shape
    assert embed == EMBED and n_rows == NUM_WORKERS * VB
    tableL = jnp.reshape(table, (vocab // 2, 2 * embed))
    out4 = _gather_out(src.T, tableL)
    return jnp.reshape(
        jnp.transpose(out4, (2, 4, 0, 1, 3)), (n_rows, row_len, embed)
    )


# speedup vs baseline: 1.4593x; 1.2290x over previous
import functools

import jax
import jax.numpy as jnp
from jax import lax
from jax.experimental import pallas as pl
from jax.experimental.pallas import tpu as pltpu
from jax.experimental.pallas import tpu_sc as plsc

EMBED = 64
FACTOR = 8.0

NUM_CORES = 2
NUM_SUBCORES = 16
NUM_WORKERS = NUM_CORES * NUM_SUBCORES
LANES = 16
VB = 128

LINEAR = pltpu.CompilerParams(
    use_tc_tiling_on_sc=False, needs_layout_passes=False
)


@jax.jit
def _gather_out(srcT, table):
    row_len, n_rows = srcT.shape
    assert n_rows == NUM_WORKERS * VB and row_len % 2 == 0
    mesh = plsc.VectorSubcoreMesh(core_axis_name="c", subcore_axis_name="s")

    @functools.partial(
        pl.kernel,
        out_type=jax.ShapeDtypeStruct(
            (row_len, EMBED // 8, n_rows // VB, 8, VB), jnp.float32),
        mesh=mesh,
        scratch_types=[
            pltpu.VMEM((row_len, VB), jnp.int32),
            pltpu.VMEM((VB, EMBED), jnp.float32),
            pltpu.VMEM((VB, EMBED), jnp.float32),
            pltpu.VMEM((EMBED // 8, 8, VB + 1), jnp.float32),
            pltpu.VMEM((EMBED // 8, 8, VB + 1), jnp.float32),
            pltpu.SemaphoreType.DMA,
            pltpu.SemaphoreType.DMA,
            pltpu.SemaphoreType.DMA,
            pltpu.SemaphoreType.DMA,
            pltpu.SemaphoreType.DMA,
        ],
        compiler_params=LINEAR,
    )
    def body(table_hbm, srcT_hbm, out4_hbm, idxT, g0, g1, w0, w1,
             isem, gs0, gs1, ws0, ws1):
        wid = lax.axis_index("s") * NUM_CORES + lax.axis_index("c")
        col0 = wid * VB
        iota = lax.iota(jnp.int32, LANES)

        pltpu.async_copy(srcT_hbm.at[:, pl.ds(col0, VB)], idxT, isem)
        pltpu.make_async_copy(
            srcT_hbm.at[:, pl.ds(0, VB)], idxT, isem
        ).wait()

        r1 = [lax.shift_right_logical(iota + 16 * k, 3) for k in range(4)]
        r2 = [lax.bitwise_and(iota + 16 * k, 7) for k in range(4)]

        def fire_gather(c, gbuf, gsem):
            pltpu.async_copy(table_hbm.at[idxT.at[c]], gbuf, gsem)

        def drain_gather(gbuf, gsem):
            pltpu.make_async_copy(
                table_hbm.at[pl.ds(0, VB)], gbuf, gsem
            ).wait()

        def build(c, gbuf, wbuf):
            def step_t(t, c2):
                colt = jnp.full((LANES,), 16 * t, jnp.int32)
                for j in range(LANES):
                    b = 16 * t + j
                    vs = [
                        gbuf[b, pl.ds(16 * k, 16)]
                        for k in range(EMBED // LANES)
                    ]
                    colv = colt + j
                    for k in range(EMBED // LANES):
                        plsc.store_scatter(
                            wbuf, (r1[k], r2[k], colv), vs[k] * FACTOR
                        )
                return c2

            lax.fori_loop(0, 8, step_t, 0)

        def fire_write(c, wbuf, wsem):
            pltpu.async_copy(
                wbuf.at[:, :, pl.ds(0, VB)], out4_hbm.at[c, :, wid], wsem
            )

        def drain_write(wbuf, wsem):
            pltpu.make_async_copy(
                wbuf.at[:, :, pl.ds(0, VB)], out4_hbm.at[0, :, 0], wsem
            ).wait()

        fire_gather(0, g0, gs0)

        def step(j, carry):
            c0 = 2 * j
            c1 = c0 + 1

            drain_gather(g0, gs0)
            fire_gather(c1, g1, gs1)

            @pl.when(j > 0)
            def _():
                drain_write(w0, ws0)

            build(c0, g0, w0)
            fire_write(c0, w0, ws0)

            drain_gather(g1, gs1)

            @pl.when(c1 + 1 < row_len)
            def _():
                fire_gather(c1 + 1, g0, gs0)

            @pl.when(j > 0)
            def _():
                drain_write(w1, ws1)

            build(c1, g1, w1)
            fire_write(c1, w1, ws1)
            return carry

        lax.fori_loop(0, row_len // 2, step, 0)
        drain_write(w0, ws0)
        drain_write(w1, ws1)

    return body(table, srcT)


def kernel(src, table):
    n_rows, row_len = src.shape
    vocab, embed = table.shape
    assert embed == EMBED and n_rows == NUM_WORKERS * VB
    out4 = _gather_out(src.T, table)
    return jnp.reshape(
        jnp.transpose(out4, (2, 4, 0, 1, 3)), (n_rows, row_len, embed)
    )
